# Initial kernel scaffold; baseline (speedup 1.0000x reference)
import jax
import jax.numpy as jnp
from jax.experimental import pallas as pl


def kernel(x, W, b):
    raise NotImplementedError("write your pallas kernel here")


# speedup vs baseline: 1.8906x; 1.8906x over previous
import jax
import jax.numpy as jnp
from jax.experimental import pallas as pl
from jax.experimental.pallas import tpu as pltpu

ROWS = 32
H = 512
W_DIM = 512
C = 32
NBLK = H // ROWS
WPAD = 640


def _conv_body(xprev_ref, xcur_ref, xnext_ref, w2_ref, b_ref, out_ref, xpad_ref):
    i = pl.program_id(1)

    xpad_ref[...] = jnp.zeros_like(xpad_ref)
    xpad_ref[:, 1:ROWS + 1, 1:513] = xcur_ref[0].astype(jnp.bfloat16)

    @pl.when(i > 0)
    def _():
        xpad_ref[:, 0:1, 1:513] = xprev_ref[0][:, ROWS - 1:ROWS, :].astype(jnp.bfloat16)

    @pl.when(i < NBLK - 1)
    def _():
        xpad_ref[:, ROWS + 1:ROWS + 2, 1:513] = xnext_ref[0][:, 0:1, :].astype(jnp.bfloat16)

    xp = xpad_ref[...]
    xcol = jnp.concatenate(
        [xp[:, :, 0:512], xp[:, :, 1:513], xp[:, :, 2:514]], axis=0)
    p = jax.lax.dot_general(
        w2_ref[...], xcol, (((1,), (0,)), ((), ())),
        preferred_element_type=jnp.float32)
    out = (p[0:32, 0:ROWS, :]
           + p[32:64, 1:ROWS + 1, :]
           + p[64:96, 2:ROWS + 2, :])
    out_ref[0] = out + b_ref[...][:, None, :]


def kernel(x, W, b):
    n = x.shape[0]
    w2 = jnp.transpose(W, (2, 0, 3, 1)).reshape(96, 96).astype(jnp.bfloat16)
    bb = jnp.broadcast_to(b[:, None], (C, W_DIM))

    grid = (n, NBLK)
    xspec = lambda f: pl.BlockSpec((1, C, ROWS, W_DIM), f)
    return pl.pallas_call(
        _conv_body,
        grid=grid,
        in_specs=[
            xspec(lambda nn, ii: (nn, 0, jnp.maximum(ii - 1, 0), 0)),
            xspec(lambda nn, ii: (nn, 0, ii, 0)),
            xspec(lambda nn, ii: (nn, 0, jnp.minimum(ii + 1, NBLK - 1), 0)),
            pl.BlockSpec((96, 96), lambda nn, ii: (0, 0)),
            pl.BlockSpec((C, W_DIM), lambda nn, ii: (0, 0)),
        ],
        out_specs=pl.BlockSpec((1, C, ROWS, W_DIM), lambda nn, ii: (nn, 0, ii, 0)),
        out_shape=jax.ShapeDtypeStruct((n, C, H, W_DIM), jnp.float32),
        scratch_shapes=[pltpu.VMEM((C, ROWS + 2, WPAD), jnp.bfloat16)],
        compiler_params=pltpu.CompilerParams(
            dimension_semantics=("parallel", "parallel"),
            vmem_limit_bytes=100 * 1024 * 1024,
        ),
    )(x, x, x, w2, bb)
